# baseline (speedup 1.0000x reference)
import functools

import jax
import jax.numpy as jnp
from jax.experimental import pallas as pl
from jax.experimental.pallas import tpu as pltpu

D_MODEL = 1024
N_CLUSTERS = 64
TOP_K = 8
BASE_TEMP = 2.0
SEQ_LEN = 2048
PRED_LEN = 512

_TEMP = BASE_TEMP * (1.0 + PRED_LEN / SEQ_LEN)
_INV_TEMP = 1.0 / _TEMP


def _fused_kernel(x_ref, d_ref, xc_ref, xr_ref, aux_ref, q_buf, acc_ref,
                  *, tile_rows, n_rows, n_tiles):
    i = pl.program_id(0)
    x_t = x_ref[...]
    d = d_ref[...]

    scores_t = jax.lax.dot_general(
        d, x_t, (((1,), (1,)), ((), ())),
        preferred_element_type=jnp.float32)
    st = scores_t * _INV_TEMP

    m0 = jnp.max(st, axis=0, keepdims=True)
    e = jnp.exp(st - m0)
    q = e * (1.0 / jnp.sum(e, axis=0, keepdims=True))
    q_buf[:, pl.ds(i * tile_rows, tile_rows)] = q

    @pl.when(i == 0)
    def _():
        acc_ref[...] = q

    @pl.when(i > 0)
    def _():
        acc_ref[...] = acc_ref[...] + q

    k = st.shape[0]
    iota = jax.lax.broadcasted_iota(jnp.int32, st.shape, 0)
    sel = jnp.zeros(st.shape, dtype=jnp.bool_)
    work = st
    neg_inf = jnp.float32(-jnp.inf)
    for _r in range(TOP_K):
        m = jnp.max(work, axis=0, keepdims=True)
        is_m = work == m
        idx = jnp.min(jnp.where(is_m, iota, k), axis=0, keepdims=True)
        first = iota == idx
        sel = jnp.logical_or(sel, first)
        work = jnp.where(first, neg_inf, work)

    ew = jnp.where(sel, e, 0.0)
    w = ew * (1.0 / jnp.sum(ew, axis=0, keepdims=True))

    xc = jax.lax.dot_general(
        w, d, (((0,), (0,)), ((), ())),
        preferred_element_type=jnp.float32)
    xc_ref[...] = xc
    xr_ref[...] = x_t - xc

    @pl.when(i == n_tiles - 1)
    def _():
        qf = q_buf[...]
        csum = jnp.sum(acc_ref[...], axis=1, keepdims=True)
        weight = (qf * qf) / csum
        rowsum = jnp.sum(weight, axis=0, keepdims=True)
        p = weight / rowsum
        lq = jnp.log(qf)
        kl_elem = p * (lq - jnp.log(csum) - jnp.log(rowsum))
        kl = jnp.sum(kl_elem) / n_rows

        gram = jax.lax.dot_general(
            d, d, (((1,), (1,)), ((), ())),
            preferred_element_type=jnp.float32)
        kk = gram.shape[0]
        r_i = jax.lax.broadcasted_iota(jnp.int32, gram.shape, 0)
        c_i = jax.lax.broadcasted_iota(jnp.int32, gram.shape, 1)
        eye = jnp.where(r_i == c_i, 1.0, 0.0).astype(gram.dtype)
        diff = gram - eye
        ortho = jnp.sum(diff * diff) / (kk * kk)

        aux_ref[0, 0] = kl * (SEQ_LEN / PRED_LEN) + 0.1 * ortho


def kernel(x, dictionary):
    B, N, D = x.shape
    K = dictionary.shape[0]
    n_rows = B * N
    tile_rows = 256
    n_tiles = n_rows // tile_rows
    xf = x.reshape(n_rows, D)

    out_types = (
        jax.ShapeDtypeStruct((n_rows, D), jnp.float32),
        jax.ShapeDtypeStruct((n_rows, D), jnp.float32),
        jax.ShapeDtypeStruct((1, 1), jnp.float32),
    )
    xc, xr, aux = pl.pallas_call(
        functools.partial(_fused_kernel, tile_rows=tile_rows,
                          n_rows=n_rows, n_tiles=n_tiles),
        grid=(n_tiles,),
        in_specs=[
            pl.BlockSpec((tile_rows, D), lambda i: (i, 0)),
            pl.BlockSpec((K, D), lambda i: (0, 0)),
        ],
        out_specs=(
            pl.BlockSpec((tile_rows, D), lambda i: (i, 0)),
            pl.BlockSpec((tile_rows, D), lambda i: (i, 0)),
            pl.BlockSpec(memory_space=pltpu.SMEM),
        ),
        out_shape=out_types,
        scratch_shapes=[
            pltpu.VMEM((K, n_rows), jnp.float32),
            pltpu.VMEM((K, tile_rows), jnp.float32),
        ],
    )(xf, dictionary)

    return (xc.reshape(B, N, D), xr.reshape(B, N, D), aux[0, 0])


# speedup vs baseline: 1.3768x; 1.3768x over previous
import functools

import jax
import jax.numpy as jnp
from jax.experimental import pallas as pl
from jax.experimental.pallas import tpu as pltpu

D_MODEL = 1024
N_CLUSTERS = 64
TOP_K = 8
BASE_TEMP = 2.0
SEQ_LEN = 2048
PRED_LEN = 512

_TEMP = BASE_TEMP * (1.0 + PRED_LEN / SEQ_LEN)
_INV_TEMP = 1.0 / _TEMP


def _fused_kernel(x_ref, d_ref, xc_ref, xr_ref, aux_ref, q_buf, acc_ref,
                  *, tile_rows, n_rows, n_tiles):
    i = pl.program_id(0)
    x_t = x_ref[...]
    d = d_ref[...]

    scores_t = jax.lax.dot_general(
        d, x_t, (((1,), (1,)), ((), ())),
        preferred_element_type=jnp.float32)
    st = scores_t * _INV_TEMP

    m0 = jnp.max(st, axis=0, keepdims=True)
    e = jnp.exp(st - m0)
    q = e * (1.0 / jnp.sum(e, axis=0, keepdims=True))
    q_buf[:, pl.ds(i * tile_rows, tile_rows)] = q

    @pl.when(i == 0)
    def _():
        acc_ref[...] = q

    @pl.when(i > 0)
    def _():
        acc_ref[...] = acc_ref[...] + q

    k = st.shape[0]
    iota = jax.lax.broadcasted_iota(jnp.int32, st.shape, 0)
    sel = jnp.zeros(st.shape, dtype=jnp.bool_)
    work = st
    neg_inf = jnp.float32(-jnp.inf)
    for _r in range(TOP_K):
        m = jnp.max(work, axis=0, keepdims=True)
        is_m = work == m
        idx = jnp.min(jnp.where(is_m, iota, k), axis=0, keepdims=True)
        first = iota == idx
        sel = jnp.logical_or(sel, first)
        work = jnp.where(first, neg_inf, work)

    ew = jnp.where(sel, e, 0.0)
    w = ew * (1.0 / jnp.sum(ew, axis=0, keepdims=True))

    xc = jax.lax.dot_general(
        w, d, (((0,), (0,)), ((), ())),
        preferred_element_type=jnp.float32)
    xc_ref[...] = xc
    xr_ref[...] = x_t - xc

    @pl.when(i == n_tiles - 1)
    def _():
        qf = q_buf[...]
        csum = jnp.sum(acc_ref[...], axis=1, keepdims=True)
        weight = (qf * qf) / csum
        rowsum = jnp.sum(weight, axis=0, keepdims=True)
        p = weight / rowsum
        lq = jnp.log(qf)
        kl_elem = p * (lq - jnp.log(csum) - jnp.log(rowsum))
        kl = jnp.sum(kl_elem) / n_rows

        gram = jax.lax.dot_general(
            d, d, (((1,), (1,)), ((), ())),
            preferred_element_type=jnp.float32)
        kk = gram.shape[0]
        r_i = jax.lax.broadcasted_iota(jnp.int32, gram.shape, 0)
        c_i = jax.lax.broadcasted_iota(jnp.int32, gram.shape, 1)
        eye = jnp.where(r_i == c_i, 1.0, 0.0).astype(gram.dtype)
        diff = gram - eye
        ortho = jnp.sum(diff * diff) / (kk * kk)

        aux_ref[0, 0] = kl * (SEQ_LEN / PRED_LEN) + 0.1 * ortho


def kernel(x, dictionary):
    B, N, D = x.shape
    K = dictionary.shape[0]
    n_rows = B * N
    tile_rows = 1024
    n_tiles = n_rows // tile_rows
    xf = x.reshape(n_rows, D)

    out_types = (
        jax.ShapeDtypeStruct((n_rows, D), jnp.float32),
        jax.ShapeDtypeStruct((n_rows, D), jnp.float32),
        jax.ShapeDtypeStruct((1, 1), jnp.float32),
    )
    xc, xr, aux = pl.pallas_call(
        functools.partial(_fused_kernel, tile_rows=tile_rows,
                          n_rows=n_rows, n_tiles=n_tiles),
        grid=(n_tiles,),
        in_specs=[
            pl.BlockSpec((tile_rows, D), lambda i: (i, 0)),
            pl.BlockSpec((K, D), lambda i: (0, 0)),
        ],
        out_specs=(
            pl.BlockSpec((tile_rows, D), lambda i: (i, 0)),
            pl.BlockSpec((tile_rows, D), lambda i: (i, 0)),
            pl.BlockSpec(memory_space=pltpu.SMEM),
        ),
        out_shape=out_types,
        scratch_shapes=[
            pltpu.VMEM((K, n_rows), jnp.float32),
            pltpu.VMEM((K, tile_rows), jnp.float32),
        ],
    )(xf, dictionary)

    return (xc.reshape(B, N, D), xr.reshape(B, N, D), aux[0, 0])


# speedup vs baseline: 1.3924x; 1.0113x over previous
import functools

import jax
import jax.numpy as jnp
from jax.experimental import pallas as pl
from jax.experimental.pallas import tpu as pltpu

D_MODEL = 1024
N_CLUSTERS = 64
TOP_K = 8
BASE_TEMP = 2.0
SEQ_LEN = 2048
PRED_LEN = 512

_TEMP = BASE_TEMP * (1.0 + PRED_LEN / SEQ_LEN)
_INV_TEMP = 1.0 / _TEMP


def _fused_kernel(x_ref, d_ref, xc_ref, xr_ref, aux_ref, q_buf, acc_ref,
                  *, tile_rows, n_rows, n_tiles):
    i = pl.program_id(0)
    x_t = x_ref[...]
    d = d_ref[...]

    scores_t = jax.lax.dot_general(
        d, x_t, (((1,), (1,)), ((), ())),
        preferred_element_type=jnp.float32)
    st = scores_t * _INV_TEMP

    m0 = jnp.max(st, axis=0, keepdims=True)
    e = jnp.exp(st - m0)
    q = e * (1.0 / jnp.sum(e, axis=0, keepdims=True))
    q_buf[:, pl.ds(i * tile_rows, tile_rows)] = q

    @pl.when(i == 0)
    def _():
        acc_ref[...] = q

    @pl.when(i > 0)
    def _():
        acc_ref[...] = acc_ref[...] + q

    k = st.shape[0]
    iota = jax.lax.broadcasted_iota(jnp.int32, st.shape, 0)
    work = st
    neg_inf = jnp.float32(-jnp.inf)
    m = m0
    for _r in range(TOP_K):
        is_m = work == m
        idx = jnp.min(jnp.where(is_m, iota, k), axis=0, keepdims=True)
        work = jnp.where(iota == idx, neg_inf, work)
        if _r < TOP_K - 1:
            m = jnp.max(work, axis=0, keepdims=True)

    ew = jnp.where(work == neg_inf, e, 0.0)
    w = ew * (1.0 / jnp.sum(ew, axis=0, keepdims=True))

    xc = jax.lax.dot_general(
        w, d, (((0,), (0,)), ((), ())),
        preferred_element_type=jnp.float32)
    xc_ref[...] = xc
    xr_ref[...] = x_t - xc

    @pl.when(i == n_tiles - 1)
    def _():
        qf = q_buf[...]
        csum = jnp.sum(acc_ref[...], axis=1, keepdims=True)
        weight = (qf * qf) / csum
        rowsum = jnp.sum(weight, axis=0, keepdims=True)
        p = weight / rowsum
        kl_elem = p * (jnp.log(qf) - jnp.log(csum) - jnp.log(rowsum))
        kl = jnp.sum(kl_elem) / n_rows

        gram = jax.lax.dot_general(
            d, d, (((1,), (1,)), ((), ())),
            preferred_element_type=jnp.float32)
        kk = gram.shape[0]
        r_i = jax.lax.broadcasted_iota(jnp.int32, gram.shape, 0)
        c_i = jax.lax.broadcasted_iota(jnp.int32, gram.shape, 1)
        eye = jnp.where(r_i == c_i, 1.0, 0.0).astype(gram.dtype)
        diff = gram - eye
        ortho = jnp.sum(diff * diff) / (kk * kk)

        aux_ref[0, 0] = kl * (SEQ_LEN / PRED_LEN) + 0.1 * ortho


def kernel(x, dictionary):
    B, N, D = x.shape
    K = dictionary.shape[0]
    n_rows = B * N
    tile_rows = 1024
    n_tiles = n_rows // tile_rows
    xf = x.reshape(n_rows, D)

    out_types = (
        jax.ShapeDtypeStruct((n_rows, D), jnp.float32),
        jax.ShapeDtypeStruct((n_rows, D), jnp.float32),
        jax.ShapeDtypeStruct((1, 1), jnp.float32),
    )
    xc, xr, aux = pl.pallas_call(
        functools.partial(_fused_kernel, tile_rows=tile_rows,
                          n_rows=n_rows, n_tiles=n_tiles),
        grid=(n_tiles,),
        in_specs=[
            pl.BlockSpec((tile_rows, D), lambda i: (i, 0)),
            pl.BlockSpec((K, D), lambda i: (0, 0)),
        ],
        out_specs=(
            pl.BlockSpec((tile_rows, D), lambda i: (i, 0)),
            pl.BlockSpec((tile_rows, D), lambda i: (i, 0)),
            pl.BlockSpec(memory_space=pltpu.SMEM),
        ),
        out_shape=out_types,
        scratch_shapes=[
            pltpu.VMEM((K, n_rows), jnp.float32),
            pltpu.VMEM((K, tile_rows), jnp.float32),
        ],
    )(xf, dictionary)

    return (xc.reshape(B, N, D), xr.reshape(B, N, D), aux[0, 0])
